# Initial kernel scaffold; baseline (speedup 1.0000x reference)
import jax
import jax.numpy as jnp
from jax.experimental import pallas as pl


def kernel(x, edge_index, batch, W1, b1, W2, b2, W3, b3, W4, b4, W5, b5, W6, b6, Wc1, bc1, Wc2, bc2):
    raise NotImplementedError("write your pallas kernel here")


# speedup vs baseline: 4.6571x; 4.6571x over previous
import functools

import jax
import jax.numpy as jnp
from jax import lax
from jax.experimental import pallas as pl
from jax.experimental.pallas import tpu as pltpu
from jax.experimental.pallas import tpu_sc as plsc

N = 10000
G = 64
K = 30
NUM_CLASSES = 10

N_PAD = 10240
TRASH = N_PAD - 1
CHUNK = 128
NW = 32
CPT = 79
E_PAD = NW * CPT * CHUNK
NCHUNKS = E_PAD // CHUNK
ZROWS = 64
RPS = N_PAD // 16
DRNG = N_PAD // NW
LCAP = E_PAD + 2176
SLAB = 16
NSLAB = NCHUNKS // SLAB
FLUSH = 2048

_SC_PARAMS = pltpu.CompilerParams(use_tc_tiling_on_sc=False)
_SC_PARAMS_NL = pltpu.CompilerParams(
    use_tc_tiling_on_sc=False, needs_layout_passes=False)


def _mesh():
    return plsc.VectorSubcoreMesh(
        core_axis_name="c", subcore_axis_name="s", num_cores=2, num_subcores=16)


def _zero_fill(zbuf, d):
    zv = jnp.zeros((16,), jnp.float32)

    @pl.loop(0, ZROWS)
    def _(r):
        for c in range(d // 16):
            zbuf[r, pl.ds(c * 16, 16)] = zv


def _redirect(sidx, didx):
    trash_v = jnp.full((16,), TRASH, jnp.int32)
    for j in range(CHUNK // 16):
        s16 = sidx[pl.ds(j * 16, 16)]
        d16 = didx[pl.ds(j * 16, 16)]
        didx[pl.ds(j * 16, 16)] = jnp.where(s16 == d16, trash_v, d16)


@functools.cache
def _get_deg_kernel():
    return functools.partial(
        pl.kernel,
        mesh=_mesh(),
        compiler_params=_SC_PARAMS,
        out_type=jax.ShapeDtypeStruct((2, N_PAD, 16), jnp.float32),
        scratch_types=[
            pltpu.VMEM((2, CHUNK), jnp.int32),
            pltpu.VMEM((2, CHUNK), jnp.int32),
            pltpu.VMEM((CHUNK, 16), jnp.float32),
            pltpu.VMEM((ZROWS, 16), jnp.float32),
            pltpu.VMEM_SHARED((N_PAD, 16), jnp.float32),
        ],
    )(_deg_body)


def _deg_body(src_hbm, dst_hbm, out_hbm, sidx, didx, crows, zbuf, aggsh):
    core = lax.axis_index("c")
    sub = lax.axis_index("s")
    wid = core * 16 + sub

    _zero_fill(zbuf, 16)

    @pl.loop(0, RPS // ZROWS)
    def _(i):
        pltpu.sync_copy(zbuf, aggsh.at[pl.ds(sub * RPS + i * ZROWS, ZROWS)])

    e0 = jnp.where(
        lax.iota(jnp.int32, 16) == jnp.zeros((16,), jnp.int32),
        jnp.ones((16,), jnp.float32),
        jnp.zeros((16,), jnp.float32),
    )

    @pl.loop(0, CHUNK)
    def _(r):
        crows[r, pl.ds(0, 16)] = e0

    plsc.subcore_barrier()

    @pl.loop(0, CPT)
    def _(i):
        ci = wid * CPT + i
        pltpu.sync_copy(src_hbm.at[ci], sidx.at[0])
        pltpu.sync_copy(dst_hbm.at[ci], didx.at[0])
        _redirect(sidx.at[0], didx.at[0])
        pltpu.sync_copy(crows, aggsh.at[didx.at[0]], add=True)

    plsc.subcore_barrier()
    pltpu.sync_copy(
        aggsh.at[pl.ds(sub * RPS, RPS)],
        out_hbm.at[core, pl.ds(sub * RPS, RPS)],
    )


@functools.cache
def _get_scan_kernel():
    return functools.partial(
        pl.kernel,
        mesh=_mesh(),
        compiler_params=_SC_PARAMS_NL,
        out_type=[
            jax.ShapeDtypeStruct((NW, LCAP), jnp.int32),
            jax.ShapeDtypeStruct((NW, LCAP), jnp.int32),
            jax.ShapeDtypeStruct((NW, LCAP), jnp.float32),
            jax.ShapeDtypeStruct((NW, 16), jnp.int32),
        ],
        scratch_types=[
            pltpu.VMEM((SLAB, CHUNK), jnp.int32),
            pltpu.VMEM((SLAB, CHUNK), jnp.int32),
            pltpu.VMEM((FLUSH + CHUNK + 16,), jnp.int32),
            pltpu.VMEM((FLUSH + CHUNK + 16,), jnp.int32),
            pltpu.VMEM((N_PAD,), jnp.float32),
            pltpu.VMEM((CHUNK,), jnp.int32),
            pltpu.VMEM((CHUNK,), jnp.int32),
            pltpu.VMEM((CHUNK,), jnp.float32),
            pltpu.VMEM((16,), jnp.int32),
        ],
    )(_scan_body)


def _scan_body(src_hbm, dst_hbm, dinv_hbm,
               esrc_hbm, eloc_hbm, enrm_hbm, ecnt_hbm,
               sslab, dslab, vsrc, vdl, dinvb, nsrc, ndl, nnrm, ncnt):
    core = lax.axis_index("c")
    sub = lax.axis_index("s")
    wid = core * 16 + sub
    lo = wid * DRNG

    pltpu.sync_copy(dinv_hbm, dinvb)

    lo_v = jnp.full((16,), lo, jnp.int32)
    hi_v = jnp.full((16,), lo + DRNG, jnp.int32)
    iota16 = lax.iota(jnp.int32, 16)

    def compact_step(g, cnt, wpos):
        s16 = sslab[g // 8, pl.ds((g % 8) * 16, 16)]
        d16 = dslab[g // 8, pl.ds((g % 8) * 16, 16)]
        m = (d16 >= lo_v) & (d16 < hi_v)
        npc = plsc.all_reduce_population_count(m)
        nmatch = jnp.max(npc)
        plsc.store_compressed(vsrc.at[pl.ds(cnt, 16)], s16, mask=m)
        plsc.store_compressed(vdl.at[pl.ds(cnt, 16)], d16 - lo_v, mask=m)
        return cnt + nmatch, wpos

    @pl.loop(0, NSLAB, init_carry=(jnp.int32(0), jnp.int32(0)))
    def outer(k, carry):
        cnt0, wpos0 = carry
        pltpu.sync_copy(src_hbm.at[pl.ds(k * SLAB, SLAB)], sslab)
        pltpu.sync_copy(dst_hbm.at[pl.ds(k * SLAB, SLAB)], dslab)

        @pl.loop(0, SLAB * 8, init_carry=(cnt0, wpos0))
        def inner(g, c):
            cnt, wpos = c
            cnt, wpos = compact_step(g, cnt, wpos)
            flush = cnt >= FLUSH

            @pl.when(flush)
            def _():
                wp = pl.multiple_of(wpos, FLUSH)
                pltpu.sync_copy(vsrc.at[pl.ds(0, FLUSH)],
                                esrc_hbm.at[wid, pl.ds(wp, FLUSH)])
                pltpu.sync_copy(vdl.at[pl.ds(0, FLUSH)],
                                eloc_hbm.at[wid, pl.ds(wp, FLUSH)])
                for j in range(2):
                    vsrc[pl.ds(j * 16, 16)] = vsrc[pl.ds(FLUSH + j * 16, 16)]
                    vdl[pl.ds(j * 16, 16)] = vdl[pl.ds(FLUSH + j * 16, 16)]

            cnt = jnp.where(flush, cnt - FLUSH, cnt)
            wpos = jnp.where(flush, wpos + FLUSH, wpos)
            return cnt, wpos

        return inner

    cnt, wpos = outer

    pad_src = jnp.full((16,), TRASH, jnp.int32)
    pad_dl = jnp.full((16,), DRNG, jnp.int32)
    for j in range(CHUNK // 16):
        vsrc[pl.ds(cnt + j * 16, 16)] = pad_src
        vdl[pl.ds(cnt + j * 16, 16)] = pad_dl
    tail = ((cnt + CHUNK - 1) // CHUNK) * CHUNK

    @pl.when(tail > 0)
    def _():
        wp = pl.multiple_of(wpos, FLUSH)
        pltpu.sync_copy(vsrc.at[pl.ds(0, FLUSH)],
                        esrc_hbm.at[wid, pl.ds(wp, FLUSH)])
        pltpu.sync_copy(vdl.at[pl.ds(0, FLUSH)],
                        eloc_hbm.at[wid, pl.ds(wp, FLUSH)])

    nb = (wpos + tail) // CHUNK
    ncnt[pl.ds(0, 16)] = jnp.full((16,), nb, jnp.int32)
    pltpu.sync_copy(ncnt, ecnt_hbm.at[wid])

    zero16 = jnp.zeros((16,), jnp.float32)

    @pl.loop(0, nb)
    def _(b):
        bo = pl.multiple_of(b * CHUNK, CHUNK)
        pltpu.sync_copy(esrc_hbm.at[wid, pl.ds(bo, CHUNK)], nsrc)
        pltpu.sync_copy(eloc_hbm.at[wid, pl.ds(bo, CHUNK)], ndl)
        maxn = jnp.full((16,), N_PAD - 1, jnp.int32)
        for j in range(CHUNK // 16):
            s16 = nsrc[pl.ds(j * 16, 16)]
            d16 = ndl[pl.ds(j * 16, 16)] + lo_v
            ds_ = plsc.load_gather(dinvb, [s16])
            dd_ = plsc.load_gather(dinvb, [jnp.minimum(d16, maxn)])
            nnrm[pl.ds(j * 16, 16)] = jnp.where(s16 == d16, zero16, ds_ * dd_)
        pltpu.sync_copy(nnrm, enrm_hbm.at[wid, pl.ds(bo, CHUNK)])


@functools.cache
def _make_acc_kernel(d):
    def _acc_body(esrc_hbm, eloc_hbm, enrm_hbm, ecnt_hbm, h_hbm, out_hbm,
                  sbuf, lbuf, nbuf, rows, acc, cbuf, gsem):
        core = lax.axis_index("c")
        sub = lax.axis_index("s")
        wid = core * 16 + sub
        lo = wid * DRNG

        @pl.loop(0, DRNG + 16)
        def _(r):
            for c in range(d // 16):
                acc[r, pl.ds(c * 16, 16)] = jnp.zeros((16,), jnp.float32)

        pltpu.sync_copy(ecnt_hbm.at[wid], cbuf)
        nb = jnp.max(cbuf[pl.ds(0, 16)])

        @pl.loop(0, nb)
        def _(b):
            bo = pl.multiple_of(b * CHUNK, CHUNK)
            pltpu.sync_copy(esrc_hbm.at[wid, pl.ds(bo, CHUNK)],
                            sbuf.at[pl.ds(0, CHUNK)])
            pltpu.sync_copy(eloc_hbm.at[wid, pl.ds(bo, CHUNK)],
                            lbuf.at[pl.ds(0, CHUNK)])
            pltpu.sync_copy(enrm_hbm.at[wid, pl.ds(bo, CHUNK)],
                            nbuf.at[pl.ds(0, CHUNK)])
            pltpu.async_copy(h_hbm.at[sbuf.at[pl.ds(0, CHUNK)]], rows, gsem).wait()

            @pl.loop(0, CHUNK)
            def _(e):
                nv = jnp.full((16,), nbuf[pl.ds(e, 16)][0], jnp.float32)
                dl = lbuf[pl.ds(e, 16)][0]
                for c in range(d // 16):
                    upd = rows[e, pl.ds(c * 16, 16)] * nv
                    plsc.addupdate(acc.at[dl, pl.ds(c * 16, 16)], upd)

        pltpu.sync_copy(acc.at[pl.ds(0, DRNG)], out_hbm.at[pl.ds(pl.multiple_of(lo, DRNG), DRNG)])

    return functools.partial(
        pl.kernel,
        mesh=_mesh(),
        compiler_params=_SC_PARAMS_NL,
        out_type=jax.ShapeDtypeStruct((N_PAD, d), jnp.float32),
        scratch_types=[
            pltpu.VMEM((CHUNK + 16,), jnp.int32),
            pltpu.VMEM((CHUNK + 16,), jnp.int32),
            pltpu.VMEM((CHUNK + 16,), jnp.float32),
            pltpu.VMEM((CHUNK, d), jnp.float32),
            pltpu.VMEM((DRNG + 16, d), jnp.float32),
            pltpu.VMEM((16,), jnp.int32),
            pltpu.SemaphoreType.DMA,
        ],
    )(_acc_body)


@functools.cache
def _get_pool_kernel():
    return functools.partial(
        pl.kernel,
        mesh=_mesh(),
        compiler_params=_SC_PARAMS_NL,
        out_type=jax.ShapeDtypeStruct((G * K, 128), jnp.float32),
        scratch_types=[
            pltpu.VMEM((N_PAD,), jnp.int32),
            pltpu.VMEM((N_PAD,), jnp.float32),
            pltpu.VMEM((64,), jnp.int32),
            pltpu.VMEM((64, 128), jnp.float32),
            pltpu.SemaphoreType.DMA,
        ],
    )(_pool_body)


def _pool_body(batch_hbm, lastv_hbm, xc_hbm, out_hbm, bbuf, vbuf, slotbuf, rows, sem):
    core = lax.axis_index("c")
    sub = lax.axis_index("s")
    wid = core * 16 + sub
    g0 = wid * 2

    pltpu.sync_copy(batch_hbm, bbuf)
    pltpu.sync_copy(lastv_hbm, vbuf)

    trash_v = jnp.full((16,), TRASH, jnp.int32)
    for j in range(4):
        slotbuf[pl.ds(16 * j, 16)] = trash_v

    iota16 = lax.iota(jnp.int32, 16)
    zero16 = jnp.zeros((16,), jnp.int32)
    one16 = jnp.ones((16,), jnp.int32)
    g0v = jnp.full((16,), g0, jnp.int32)
    g1v = g0v + one16
    k16 = jnp.full((16,), K, jnp.int32)

    @pl.loop(0, N_PAD // 16, init_carry=(zero16, zero16, zero16))
    def counts(j, c):
        s0v, n0v, n1v = c
        b16 = bbuf[pl.ds(j * 16, 16)]
        lt = plsc.all_reduce_population_count(b16 < g0v)
        e0 = plsc.all_reduce_population_count(b16 == g0v)
        e1 = plsc.all_reduce_population_count(b16 == g1v)
        return (s0v + lt, n0v + e0, n1v + e1)

    s0v, n0v, n1v = counts
    s0 = jnp.max(s0v)
    n0 = jnp.max(n0v)
    n1 = jnp.max(n1v)

    for gl in range(2):
        start = s0 if gl == 0 else s0 + n0
        n = n0 if gl == 0 else n1
        nv = jnp.full((16,), n, jnp.int32)
        base_slot = jnp.full((16,), gl * K, jnp.int32)

        @pl.loop(0, (n + 15) // 16)
        def _(ic):
            bi = ic * 16
            ivec = iota16 + jnp.full((16,), bi, jnp.int32)
            vi = vbuf[pl.ds(start + bi, 16)]

            @pl.loop(0, n, init_carry=zero16)
            def rank_loop(j, rank):
                vj = vbuf[pl.ds(start + j, 16)][0]
                vjv = jnp.full((16,), vj, jnp.float32)
                jv = jnp.full((16,), j, jnp.int32)
                beats = (vjv > vi) | ((vjv == vi) & (jv < ivec))
                return rank + jnp.where(beats, one16, zero16)

            rank = rank_loop
            mask = (rank < k16) & (ivec < nv)
            node = jnp.full((16,), start + bi, jnp.int32) + iota16
            plsc.store_scatter(slotbuf, [base_slot + rank], node, mask=mask)

    pltpu.async_copy(xc_hbm.at[slotbuf], rows, sem).wait()
    pltpu.sync_copy(rows.at[pl.ds(0, 2 * K)], out_hbm.at[pl.ds(wid * 2 * K, 2 * K)])


def _prep_body(degP_ref, x_ref, w1_ref, dinv_ref, invdeg_ref, h1_ref):
    deg = degP_ref[0, :, 0:1] + degP_ref[1, :, 0:1] + 1.0
    dinv_ref[...] = lax.rsqrt(deg)
    invdeg_ref[...] = 1.0 / deg
    h1_ref[...] = jnp.dot(x_ref[...], w1_ref[...], preferred_element_type=jnp.float32)


def _tc_prep(degP, xpad, W1):
    return pl.pallas_call(
        _prep_body,
        out_shape=[
            jax.ShapeDtypeStruct((N_PAD, 1), jnp.float32),
            jax.ShapeDtypeStruct((N_PAD, 1), jnp.float32),
            jax.ShapeDtypeStruct((N_PAD, 32), jnp.float32),
        ],
    )(degP, xpad, W1)


def _layer_body(w, wn):
    def body(acc_ref, h_ref, invdeg_ref, b_ref, *rest):
        if wn:
            wn_ref, xl_ref, hn_ref = rest
        else:
            (xl_ref,) = rest
        rows = lax.broadcasted_iota(jnp.int32, (N_PAD, 1), 0)
        valid = jnp.where(rows < N, 1.0, 0.0)
        t = acc_ref[:, :w] + invdeg_ref[...] * h_ref[...]
        xl = jnp.tanh(t + b_ref[...]) * valid
        xl_ref[...] = xl
        if wn:
            hn_ref[...] = jnp.dot(xl, wn_ref[...], preferred_element_type=jnp.float32)
    return body


def _tc_layer(acc, h, invdeg, b, Wn):
    w = h.shape[1]
    return pl.pallas_call(
        _layer_body(w, True),
        out_shape=[
            jax.ShapeDtypeStruct((N_PAD, w), jnp.float32),
            jax.ShapeDtypeStruct((N_PAD, Wn.shape[1]), jnp.float32),
        ],
    )(acc, h, invdeg, b.reshape(1, w), Wn)


_PACK_ROWS = 1280


def _layer4_pack_body(acc_ref, h_ref, invdeg_ref, b_ref,
                      x1_ref, x2_ref, x3_ref, xc_ref, lastv_ref):
    rows = (lax.broadcasted_iota(jnp.int32, (_PACK_ROWS, 1), 0)
            + pl.program_id(0) * _PACK_ROWS)
    valid = jnp.where(rows < N, 1.0, 0.0)
    t = acc_ref[:, 0:1] + invdeg_ref[...] * h_ref[...]
    x4 = jnp.tanh(t + b_ref[...]) * valid
    z = jnp.zeros((_PACK_ROWS, 31), jnp.float32)
    xc_ref[...] = jnp.concatenate(
        [x1_ref[...], x2_ref[...], x3_ref[...], x4, z], axis=1)
    lastv_ref[...] = x4


def _tc_layer4_pack(acc, h, invdeg, b, x1, x2, x3):
    nb = N_PAD // _PACK_ROWS
    row_spec = lambda w: pl.BlockSpec((_PACK_ROWS, w), lambda i: (i, 0))
    return pl.pallas_call(
        _layer4_pack_body,
        grid=(nb,),
        in_specs=[
            row_spec(16),
            row_spec(1),
            row_spec(1),
            pl.BlockSpec((1, 1), lambda i: (0, 0)),
            row_spec(32),
            row_spec(32),
            row_spec(32),
        ],
        out_specs=[row_spec(128), row_spec(1)],
        out_shape=[
            jax.ShapeDtypeStruct((N_PAD, 128), jnp.float32),
            jax.ShapeDtypeStruct((N_PAD, 1), jnp.float32),
        ],
    )(acc, h, invdeg, b.reshape(1, 1), x1, x2, x3)


def _tail_body(dense_ref, w5_ref, b5_ref, w6_ref, b6_ref, wc1_ref, bc1_ref,
               wc2_ref, bc2_ref, out_ref):
    t1 = jnp.dot(dense_ref[...], w5_ref[...], preferred_element_type=jnp.float32)
    t1 = jax.nn.relu(t1 + b5_ref[...])
    t3 = t1.reshape(G * K // 2, 2, 16)
    p = jnp.maximum(t3[:, 0, :], t3[:, 1, :])
    p3 = p.reshape(G, 15, 16)
    blocks = []
    for pp in range(11):
        accp = jnp.zeros((G, 32), jnp.float32)
        for k in range(5):
            accp = accp + jnp.dot(p3[:, pp + k, :], w6_ref[k],
                                  preferred_element_type=jnp.float32)
        blocks.append(accp)
    acc = jnp.concatenate(blocks, axis=1)
    acc = jax.nn.relu(acc + jnp.tile(b6_ref[...], (1, 11)))
    h = jax.nn.relu(jnp.dot(acc, wc1_ref[...], preferred_element_type=jnp.float32) + bc1_ref[...])
    out_ref[...] = jnp.dot(h, wc2_ref[...], preferred_element_type=jnp.float32) + bc2_ref[...]


def _tc_tail(dense, W5t, b5, W6k, b6, Wc1r, bc1, Wc2, bc2):
    return pl.pallas_call(
        _tail_body,
        out_shape=jax.ShapeDtypeStruct((G, NUM_CLASSES), jnp.float32),
    )(dense, W5t, b5.reshape(1, 16), W6k, b6.reshape(1, 32), Wc1r,
      bc1.reshape(1, 256), Wc2, bc2.reshape(1, NUM_CLASSES))


def kernel(x, edge_index, batch, W1, b1, W2, b2, W3, b3, W4, b4, W5, b5, W6, b6, Wc1, bc1, Wc2, bc2):
    src = edge_index[0]
    dst = edge_index[1]
    epad = jnp.full((E_PAD - src.shape[0],), TRASH, jnp.int32)
    src_p = jnp.concatenate([src, epad]).reshape(NCHUNKS, CHUNK)
    dst_p = jnp.concatenate([dst, epad]).reshape(NCHUNKS, CHUNK)
    xpad = jnp.zeros((N_PAD, 128), x.dtype).at[:N].set(x)
    batch_pad = jnp.concatenate([batch, jnp.full((N_PAD - N,), 127, jnp.int32)])
    W5t = jnp.zeros((128, 16), W5.dtype).at[:97, :].set(W5[:, 0, :].T)
    W6k = W6.transpose(2, 1, 0)
    Wc1r = Wc1.reshape(32, 11, 256).transpose(1, 0, 2).reshape(352, 256)

    degP = _get_deg_kernel()(src_p, dst_p)
    dinv, invdeg, h1 = _tc_prep(degP, xpad, W1)

    esrc, eloc, enrm, ecnt = _get_scan_kernel()(src_p, dst_p, dinv.reshape(N_PAD))

    acck32 = _make_acc_kernel(32)
    A1 = acck32(esrc, eloc, enrm, ecnt, h1)
    x1, h2 = _tc_layer(A1, h1, invdeg, b1, W2)
    A2 = acck32(esrc, eloc, enrm, ecnt, h2)
    x2, h3 = _tc_layer(A2, h2, invdeg, b2, W3)
    A3 = acck32(esrc, eloc, enrm, ecnt, h3)
    x3, h4 = _tc_layer(A3, h3, invdeg, b3, W4)
    h4p = jnp.zeros((N_PAD, 16), jnp.float32).at[:, :1].set(h4)
    A4 = _make_acc_kernel(16)(esrc, eloc, enrm, ecnt, h4p)
    xc, lastv = _tc_layer4_pack(A4, h4, invdeg, b4, x1, x2, x3)

    dense = _get_pool_kernel()(batch_pad, lastv.reshape(N_PAD), xc)
    return _tc_tail(dense, W5t, b5, W6k, b6, Wc1r, bc1, Wc2, bc2)


# speedup vs baseline: 6.2095x; 1.3333x over previous
import functools

import jax
import jax.numpy as jnp
from jax import lax
from jax.experimental import pallas as pl
from jax.experimental.pallas import tpu as pltpu
from jax.experimental.pallas import tpu_sc as plsc

N = 10000
G = 64
K = 30
NUM_CLASSES = 10

N_PAD = 10240
TRASH = N_PAD - 1
CHUNK = 128
NW = 32
CPT = 79
E_PAD = NW * CPT * CHUNK
NCHUNKS = E_PAD // CHUNK
ZROWS = 64
RPS = N_PAD // 16
DRNG = N_PAD // NW
LCAP = E_PAD + 14592
CAPB = 96
SLAB = 16
NSLAB = NCHUNKS // SLAB
FLUSH = 2048

_SC_PARAMS = pltpu.CompilerParams(use_tc_tiling_on_sc=False)
_SC_PARAMS_NL = pltpu.CompilerParams(
    use_tc_tiling_on_sc=False, needs_layout_passes=False)


def _mesh():
    return plsc.VectorSubcoreMesh(
        core_axis_name="c", subcore_axis_name="s", num_cores=2, num_subcores=16)


def _zero_fill(zbuf, d):
    zv = jnp.zeros((16,), jnp.float32)

    @pl.loop(0, ZROWS)
    def _(r):
        for c in range(d // 16):
            zbuf[r, pl.ds(c * 16, 16)] = zv


def _redirect(sidx, didx):
    trash_v = jnp.full((16,), TRASH, jnp.int32)
    for j in range(CHUNK // 16):
        s16 = sidx[pl.ds(j * 16, 16)]
        d16 = didx[pl.ds(j * 16, 16)]
        didx[pl.ds(j * 16, 16)] = jnp.where(s16 == d16, trash_v, d16)


@functools.cache
def _get_deg_kernel():
    return functools.partial(
        pl.kernel,
        mesh=_mesh(),
        compiler_params=_SC_PARAMS,
        out_type=jax.ShapeDtypeStruct((2, N_PAD, 16), jnp.float32),
        scratch_types=[
            pltpu.VMEM((2, CHUNK), jnp.int32),
            pltpu.VMEM((2, CHUNK), jnp.int32),
            pltpu.VMEM((CHUNK, 16), jnp.float32),
            pltpu.VMEM((ZROWS, 16), jnp.float32),
            pltpu.VMEM_SHARED((N_PAD, 16), jnp.float32),
        ],
    )(_deg_body)


def _deg_body(src_hbm, dst_hbm, out_hbm, sidx, didx, crows, zbuf, aggsh):
    core = lax.axis_index("c")
    sub = lax.axis_index("s")
    wid = core * 16 + sub

    _zero_fill(zbuf, 16)

    @pl.loop(0, RPS // ZROWS)
    def _(i):
        pltpu.sync_copy(zbuf, aggsh.at[pl.ds(sub * RPS + i * ZROWS, ZROWS)])

    e0 = jnp.where(
        lax.iota(jnp.int32, 16) == jnp.zeros((16,), jnp.int32),
        jnp.ones((16,), jnp.float32),
        jnp.zeros((16,), jnp.float32),
    )

    @pl.loop(0, CHUNK)
    def _(r):
        crows[r, pl.ds(0, 16)] = e0

    plsc.subcore_barrier()

    @pl.loop(0, CPT)
    def _(i):
        ci = wid * CPT + i
        pltpu.sync_copy(src_hbm.at[ci], sidx.at[0])
        pltpu.sync_copy(dst_hbm.at[ci], didx.at[0])
        _redirect(sidx.at[0], didx.at[0])
        pltpu.sync_copy(crows, aggsh.at[didx.at[0]], add=True)

    plsc.subcore_barrier()
    pltpu.sync_copy(
        aggsh.at[pl.ds(sub * RPS, RPS)],
        out_hbm.at[core, pl.ds(sub * RPS, RPS)],
    )


@functools.cache
def _get_scan_kernel():
    return functools.partial(
        pl.kernel,
        mesh=_mesh(),
        compiler_params=_SC_PARAMS_NL,
        out_type=[
            jax.ShapeDtypeStruct((NW, LCAP), jnp.int32),
            jax.ShapeDtypeStruct((NW, LCAP), jnp.int32),
            jax.ShapeDtypeStruct((NW, LCAP), jnp.float32),
            jax.ShapeDtypeStruct((NW, 16), jnp.int32),
        ],
        scratch_types=[
            pltpu.VMEM((SLAB, CHUNK), jnp.int32),
            pltpu.VMEM((SLAB, CHUNK), jnp.int32),
            pltpu.VMEM((FLUSH + CHUNK + 16,), jnp.int32),
            pltpu.VMEM((FLUSH + CHUNK + 16,), jnp.int32),
            pltpu.VMEM((N_PAD,), jnp.float32),
            pltpu.VMEM((CHUNK,), jnp.int32),
            pltpu.VMEM((CHUNK,), jnp.int32),
            pltpu.VMEM((CHUNK,), jnp.float32),
            pltpu.VMEM((16,), jnp.int32),
        ],
    )(_scan_body)


def _scan_body(src_hbm, dst_hbm, dinv_hbm,
               esrc_hbm, eloc_hbm, enrm_hbm, ecnt_hbm,
               sslab, dslab, vsrc, vdl, dinvb, nsrc, ndl, nnrm, ncnt):
    core = lax.axis_index("c")
    sub = lax.axis_index("s")
    wid = core * 16 + sub
    lo = wid * DRNG

    pltpu.sync_copy(dinv_hbm, dinvb)

    lo_v = jnp.full((16,), lo, jnp.int32)
    hi_v = jnp.full((16,), lo + DRNG, jnp.int32)
    iota16 = lax.iota(jnp.int32, 16)

    def compact_step(g, cnt, wpos):
        s16 = sslab[g // 8, pl.ds((g % 8) * 16, 16)]
        d16 = dslab[g // 8, pl.ds((g % 8) * 16, 16)]
        m = (d16 >= lo_v) & (d16 < hi_v)
        npc = plsc.all_reduce_population_count(m)
        nmatch = jnp.max(npc)
        plsc.store_compressed(vsrc.at[pl.ds(cnt, 16)], s16, mask=m)
        plsc.store_compressed(vdl.at[pl.ds(cnt, 16)], d16 - lo_v, mask=m)
        return cnt + nmatch, wpos

    @pl.loop(0, NSLAB, init_carry=(jnp.int32(0), jnp.int32(0)))
    def outer(k, carry):
        cnt0, wpos0 = carry
        pltpu.sync_copy(src_hbm.at[pl.ds(k * SLAB, SLAB)], sslab)
        pltpu.sync_copy(dst_hbm.at[pl.ds(k * SLAB, SLAB)], dslab)

        @pl.loop(0, SLAB * 8, init_carry=(cnt0, wpos0))
        def inner(g, c):
            cnt, wpos = c
            cnt, wpos = compact_step(g, cnt, wpos)
            flush = cnt >= FLUSH

            @pl.when(flush)
            def _():
                wp = pl.multiple_of(wpos, FLUSH)
                pltpu.sync_copy(vsrc.at[pl.ds(0, FLUSH)],
                                esrc_hbm.at[wid, pl.ds(wp, FLUSH)])
                pltpu.sync_copy(vdl.at[pl.ds(0, FLUSH)],
                                eloc_hbm.at[wid, pl.ds(wp, FLUSH)])
                for j in range(2):
                    vsrc[pl.ds(j * 16, 16)] = vsrc[pl.ds(FLUSH + j * 16, 16)]
                    vdl[pl.ds(j * 16, 16)] = vdl[pl.ds(FLUSH + j * 16, 16)]

            cnt = jnp.where(flush, cnt - FLUSH, cnt)
            wpos = jnp.where(flush, wpos + FLUSH, wpos)
            return cnt, wpos

        return inner

    cnt, wpos = outer

    pad_src = jnp.full((16,), TRASH, jnp.int32)
    pad_dl = jnp.full((16,), DRNG, jnp.int32)
    for j in range(CHUNK // 16):
        vsrc[pl.ds(cnt + j * 16, 16)] = pad_src
        vdl[pl.ds(cnt + j * 16, 16)] = pad_dl
    tail = ((cnt + CHUNK - 1) // CHUNK) * CHUNK

    @pl.when(tail > 0)
    def _():
        wp = pl.multiple_of(wpos, FLUSH)
        pltpu.sync_copy(vsrc.at[pl.ds(0, FLUSH)],
                        esrc_hbm.at[wid, pl.ds(wp, FLUSH)])
        pltpu.sync_copy(vdl.at[pl.ds(0, FLUSH)],
                        eloc_hbm.at[wid, pl.ds(wp, FLUSH)])

    nb = (wpos + tail) // CHUNK
    ncnt[pl.ds(0, 16)] = jnp.full((16,), nb, jnp.int32)
    pltpu.sync_copy(ncnt, ecnt_hbm.at[wid])

    zero16 = jnp.zeros((16,), jnp.float32)

    @pl.loop(0, nb)
    def _(b):
        bo = pl.multiple_of(b * CHUNK, CHUNK)
        pltpu.sync_copy(esrc_hbm.at[wid, pl.ds(bo, CHUNK)], nsrc)
        pltpu.sync_copy(eloc_hbm.at[wid, pl.ds(bo, CHUNK)], ndl)
        maxn = jnp.full((16,), N_PAD - 1, jnp.int32)
        for j in range(CHUNK // 16):
            s16 = nsrc[pl.ds(j * 16, 16)]
            d16 = ndl[pl.ds(j * 16, 16)] + lo_v
            ds_ = plsc.load_gather(dinvb, [s16])
            dd_ = plsc.load_gather(dinvb, [jnp.minimum(d16, maxn)])
            nnrm[pl.ds(j * 16, 16)] = jnp.where(s16 == d16, zero16, ds_ * dd_)
        pltpu.sync_copy(nnrm, enrm_hbm.at[wid, pl.ds(bo, CHUNK)])


@functools.cache
def _make_acc_kernel(d):
    def _acc_body(esrc_hbm, eloc_hbm, enrm_hbm, ecnt_hbm, h_hbm, out_hbm,
                  lsrc, lloc, lnrm, rows0, rows1, acc, cbuf, gs0, gs1):
        core = lax.axis_index("c")
        sub = lax.axis_index("s")
        wid = core * 16 + sub
        lo = wid * DRNG

        @pl.loop(0, DRNG + 16)
        def _(r):
            for c in range(d // 16):
                acc[r, pl.ds(c * 16, 16)] = jnp.zeros((16,), jnp.float32)

        pltpu.sync_copy(ecnt_hbm.at[wid], cbuf)
        nb = jnp.max(cbuf[pl.ds(0, 16)])
        nsb = (nb + CAPB - 1) // CAPB

        def start_gather(b, rows, sem):
            io = pl.multiple_of(b * CHUNK, CHUNK)
            return pltpu.async_copy(
                h_hbm.at[lsrc.at[pl.ds(io, CHUNK)]], rows, sem)

        def wait_gather(b, rows, sem):
            io = pl.multiple_of(b * CHUNK, CHUNK)
            pltpu.make_async_copy(
                h_hbm.at[lsrc.at[pl.ds(io, CHUNK)]], rows, sem).wait()

        def process(b, rows):
            @pl.loop(0, CHUNK, unroll=4)
            def _(e):
                off = b * CHUNK + e
                nv = jnp.full((16,), lnrm[pl.ds(off, 16)][0], jnp.float32)
                dl = lloc[pl.ds(off, 16)][0]
                for c in range(d // 16):
                    upd = rows[e, pl.ds(c * 16, 16)] * nv
                    plsc.addupdate(acc.at[dl, pl.ds(c * 16, 16)], upd)

        @pl.loop(0, nsb)
        def _(sb):
            base = sb * CAPB
            rem = jnp.minimum(nb - base, CAPB)
            bo = pl.multiple_of(base * CHUNK, CHUNK)
            pltpu.sync_copy(esrc_hbm.at[wid, pl.ds(bo, CAPB * CHUNK)],
                            lsrc.at[pl.ds(0, CAPB * CHUNK)])
            pltpu.sync_copy(eloc_hbm.at[wid, pl.ds(bo, CAPB * CHUNK)],
                            lloc.at[pl.ds(0, CAPB * CHUNK)])
            pltpu.sync_copy(enrm_hbm.at[wid, pl.ds(bo, CAPB * CHUNK)],
                            lnrm.at[pl.ds(0, CAPB * CHUNK)])

            @pl.when(rem > 0)
            def _():
                start_gather(0, rows0, gs0)

            @pl.loop(0, rem, step=2)
            def _(b0):
                @pl.when(b0 + 1 < rem)
                def _():
                    start_gather(b0 + 1, rows1, gs1)

                wait_gather(b0, rows0, gs0)
                process(b0, rows0)

                @pl.when(b0 + 2 < rem)
                def _():
                    start_gather(b0 + 2, rows0, gs0)

                @pl.when(b0 + 1 < rem)
                def _():
                    wait_gather(b0 + 1, rows1, gs1)
                    process(b0 + 1, rows1)

        pltpu.sync_copy(acc.at[pl.ds(0, DRNG)], out_hbm.at[pl.ds(pl.multiple_of(lo, DRNG), DRNG)])

    return functools.partial(
        pl.kernel,
        mesh=_mesh(),
        compiler_params=_SC_PARAMS_NL,
        out_type=jax.ShapeDtypeStruct((N_PAD, d), jnp.float32),
        scratch_types=[
            pltpu.VMEM((CAPB * CHUNK + 16,), jnp.int32),
            pltpu.VMEM((CAPB * CHUNK + 16,), jnp.int32),
            pltpu.VMEM((CAPB * CHUNK + 16,), jnp.float32),
            pltpu.VMEM((CHUNK, d), jnp.float32),
            pltpu.VMEM((CHUNK, d), jnp.float32),
            pltpu.VMEM((DRNG + 16, d), jnp.float32),
            pltpu.VMEM((16,), jnp.int32),
            pltpu.SemaphoreType.DMA,
            pltpu.SemaphoreType.DMA,
        ],
    )(_acc_body)


@functools.cache
def _get_pool_kernel():
    return functools.partial(
        pl.kernel,
        mesh=_mesh(),
        compiler_params=_SC_PARAMS_NL,
        out_type=jax.ShapeDtypeStruct((G * K, 128), jnp.float32),
        scratch_types=[
            pltpu.VMEM((N_PAD,), jnp.int32),
            pltpu.VMEM((N_PAD,), jnp.float32),
            pltpu.VMEM((64,), jnp.int32),
            pltpu.VMEM((64, 128), jnp.float32),
            pltpu.SemaphoreType.DMA,
        ],
    )(_pool_body)


def _pool_body(batch_hbm, lastv_hbm, xc_hbm, out_hbm, bbuf, vbuf, slotbuf, rows, sem):
    core = lax.axis_index("c")
    sub = lax.axis_index("s")
    wid = core * 16 + sub
    g0 = wid * 2

    pltpu.sync_copy(batch_hbm, bbuf)
    pltpu.sync_copy(lastv_hbm, vbuf)

    trash_v = jnp.full((16,), TRASH, jnp.int32)
    for j in range(4):
        slotbuf[pl.ds(16 * j, 16)] = trash_v

    iota16 = lax.iota(jnp.int32, 16)
    zero16 = jnp.zeros((16,), jnp.int32)
    one16 = jnp.ones((16,), jnp.int32)
    g0v = jnp.full((16,), g0, jnp.int32)
    g1v = g0v + one16
    k16 = jnp.full((16,), K, jnp.int32)

    @pl.loop(0, N_PAD // 16, init_carry=(zero16, zero16, zero16))
    def counts(j, c):
        s0v, n0v, n1v = c
        b16 = bbuf[pl.ds(j * 16, 16)]
        lt = plsc.all_reduce_population_count(b16 < g0v)
        e0 = plsc.all_reduce_population_count(b16 == g0v)
        e1 = plsc.all_reduce_population_count(b16 == g1v)
        return (s0v + lt, n0v + e0, n1v + e1)

    s0v, n0v, n1v = counts
    s0 = jnp.max(s0v)
    n0 = jnp.max(n0v)
    n1 = jnp.max(n1v)

    for gl in range(2):
        start = s0 if gl == 0 else s0 + n0
        n = n0 if gl == 0 else n1
        nv = jnp.full((16,), n, jnp.int32)
        base_slot = jnp.full((16,), gl * K, jnp.int32)

        @pl.loop(0, (n + 15) // 16)
        def _(ic):
            bi = ic * 16
            ivec = iota16 + jnp.full((16,), bi, jnp.int32)
            vi = vbuf[pl.ds(start + bi, 16)]

            @pl.loop(0, n, init_carry=zero16)
            def rank_loop(j, rank):
                vj = vbuf[pl.ds(start + j, 16)][0]
                vjv = jnp.full((16,), vj, jnp.float32)
                jv = jnp.full((16,), j, jnp.int32)
                beats = (vjv > vi) | ((vjv == vi) & (jv < ivec))
                return rank + jnp.where(beats, one16, zero16)

            rank = rank_loop
            mask = (rank < k16) & (ivec < nv)
            node = jnp.full((16,), start + bi, jnp.int32) + iota16
            plsc.store_scatter(slotbuf, [base_slot + rank], node, mask=mask)

    pltpu.async_copy(xc_hbm.at[slotbuf], rows, sem).wait()
    pltpu.sync_copy(rows.at[pl.ds(0, 2 * K)], out_hbm.at[pl.ds(wid * 2 * K, 2 * K)])


def _prep_body(degP_ref, x_ref, w1_ref, dinv_ref, invdeg_ref, h1_ref):
    deg = degP_ref[0, :, 0:1] + degP_ref[1, :, 0:1] + 1.0
    dinv_ref[...] = lax.rsqrt(deg)
    invdeg_ref[...] = 1.0 / deg
    h1_ref[...] = jnp.dot(x_ref[...], w1_ref[...], preferred_element_type=jnp.float32)


def _tc_prep(degP, xpad, W1):
    return pl.pallas_call(
        _prep_body,
        out_shape=[
            jax.ShapeDtypeStruct((N_PAD, 1), jnp.float32),
            jax.ShapeDtypeStruct((N_PAD, 1), jnp.float32),
            jax.ShapeDtypeStruct((N_PAD, 32), jnp.float32),
        ],
    )(degP, xpad, W1)


def _layer_body(w, wn):
    def body(acc_ref, h_ref, invdeg_ref, b_ref, *rest):
        if wn:
            wn_ref, xl_ref, hn_ref = rest
        else:
            (xl_ref,) = rest
        rows = lax.broadcasted_iota(jnp.int32, (N_PAD, 1), 0)
        valid = jnp.where(rows < N, 1.0, 0.0)
        t = acc_ref[:, :w] + invdeg_ref[...] * h_ref[...]
        xl = jnp.tanh(t + b_ref[...]) * valid
        xl_ref[...] = xl
        if wn:
            hn_ref[...] = jnp.dot(xl, wn_ref[...], preferred_element_type=jnp.float32)
    return body


def _tc_layer(acc, h, invdeg, b, Wn):
    w = h.shape[1]
    return pl.pallas_call(
        _layer_body(w, True),
        out_shape=[
            jax.ShapeDtypeStruct((N_PAD, w), jnp.float32),
            jax.ShapeDtypeStruct((N_PAD, Wn.shape[1]), jnp.float32),
        ],
    )(acc, h, invdeg, b.reshape(1, w), Wn)


_PACK_ROWS = 1280


def _layer4_pack_body(acc_ref, h_ref, invdeg_ref, b_ref,
                      x1_ref, x2_ref, x3_ref, xc_ref, lastv_ref):
    rows = (lax.broadcasted_iota(jnp.int32, (_PACK_ROWS, 1), 0)
            + pl.program_id(0) * _PACK_ROWS)
    valid = jnp.where(rows < N, 1.0, 0.0)
    t = acc_ref[:, 0:1] + invdeg_ref[...] * h_ref[...]
    x4 = jnp.tanh(t + b_ref[...]) * valid
    z = jnp.zeros((_PACK_ROWS, 31), jnp.float32)
    xc_ref[...] = jnp.concatenate(
        [x1_ref[...], x2_ref[...], x3_ref[...], x4, z], axis=1)
    lastv_ref[...] = x4


def _tc_layer4_pack(acc, h, invdeg, b, x1, x2, x3):
    nb = N_PAD // _PACK_ROWS
    row_spec = lambda w: pl.BlockSpec((_PACK_ROWS, w), lambda i: (i, 0))
    return pl.pallas_call(
        _layer4_pack_body,
        grid=(nb,),
        in_specs=[
            row_spec(16),
            row_spec(1),
            row_spec(1),
            pl.BlockSpec((1, 1), lambda i: (0, 0)),
            row_spec(32),
            row_spec(32),
            row_spec(32),
        ],
        out_specs=[row_spec(128), row_spec(1)],
        out_shape=[
            jax.ShapeDtypeStruct((N_PAD, 128), jnp.float32),
            jax.ShapeDtypeStruct((N_PAD, 1), jnp.float32),
        ],
    )(acc, h, invdeg, b.reshape(1, 1), x1, x2, x3)


def _tail_body(dense_ref, w5_ref, b5_ref, w6_ref, b6_ref, wc1_ref, bc1_ref,
               wc2_ref, bc2_ref, out_ref):
    t1 = jnp.dot(dense_ref[...], w5_ref[...], preferred_element_type=jnp.float32)
    t1 = jax.nn.relu(t1 + b5_ref[...])
    t3 = t1.reshape(G * K // 2, 2, 16)
    p = jnp.maximum(t3[:, 0, :], t3[:, 1, :])
    p3 = p.reshape(G, 15, 16)
    blocks = []
    for pp in range(11):
        accp = jnp.zeros((G, 32), jnp.float32)
        for k in range(5):
            accp = accp + jnp.dot(p3[:, pp + k, :], w6_ref[k],
                                  preferred_element_type=jnp.float32)
        blocks.append(accp)
    acc = jnp.concatenate(blocks, axis=1)
    acc = jax.nn.relu(acc + jnp.tile(b6_ref[...], (1, 11)))
    h = jax.nn.relu(jnp.dot(acc, wc1_ref[...], preferred_element_type=jnp.float32) + bc1_ref[...])
    out_ref[...] = jnp.dot(h, wc2_ref[...], preferred_element_type=jnp.float32) + bc2_ref[...]


def _tc_tail(dense, W5t, b5, W6k, b6, Wc1r, bc1, Wc2, bc2):
    return pl.pallas_call(
        _tail_body,
        out_shape=jax.ShapeDtypeStruct((G, NUM_CLASSES), jnp.float32),
    )(dense, W5t, b5.reshape(1, 16), W6k, b6.reshape(1, 32), Wc1r,
      bc1.reshape(1, 256), Wc2, bc2.reshape(1, NUM_CLASSES))


def kernel(x, edge_index, batch, W1, b1, W2, b2, W3, b3, W4, b4, W5, b5, W6, b6, Wc1, bc1, Wc2, bc2):
    src = edge_index[0]
    dst = edge_index[1]
    epad = jnp.full((E_PAD - src.shape[0],), TRASH, jnp.int32)
    src_p = jnp.concatenate([src, epad]).reshape(NCHUNKS, CHUNK)
    dst_p = jnp.concatenate([dst, epad]).reshape(NCHUNKS, CHUNK)
    xpad = jnp.zeros((N_PAD, 128), x.dtype).at[:N].set(x)
    batch_pad = jnp.concatenate([batch, jnp.full((N_PAD - N,), 127, jnp.int32)])
    W5t = jnp.zeros((128, 16), W5.dtype).at[:97, :].set(W5[:, 0, :].T)
    W6k = W6.transpose(2, 1, 0)
    Wc1r = Wc1.reshape(32, 11, 256).transpose(1, 0, 2).reshape(352, 256)

    degP = _get_deg_kernel()(src_p, dst_p)
    dinv, invdeg, h1 = _tc_prep(degP, xpad, W1)

    esrc, eloc, enrm, ecnt = _get_scan_kernel()(src_p, dst_p, dinv.reshape(N_PAD))

    acck32 = _make_acc_kernel(32)
    A1 = acck32(esrc, eloc, enrm, ecnt, h1)
    x1, h2 = _tc_layer(A1, h1, invdeg, b1, W2)
    A2 = acck32(esrc, eloc, enrm, ecnt, h2)
    x2, h3 = _tc_layer(A2, h2, invdeg, b2, W3)
    A3 = acck32(esrc, eloc, enrm, ecnt, h3)
    x3, h4 = _tc_layer(A3, h3, invdeg, b3, W4)
    h4p = jnp.zeros((N_PAD, 16), jnp.float32).at[:, :1].set(h4)
    A4 = _make_acc_kernel(16)(esrc, eloc, enrm, ecnt, h4p)
    xc, lastv = _tc_layer4_pack(A4, h4, invdeg, b4, x1, x2, x3)

    dense = _get_pool_kernel()(batch_pad, lastv.reshape(N_PAD), xc)
    return _tc_tail(dense, W5t, b5, W6k, b6, Wc1r, bc1, Wc2, bc2)


# speedup vs baseline: 6.2127x; 1.0005x over previous
import functools

import jax
import jax.numpy as jnp
from jax import lax
from jax.experimental import pallas as pl
from jax.experimental.pallas import tpu as pltpu
from jax.experimental.pallas import tpu_sc as plsc

N = 10000
G = 64
K = 30
NUM_CLASSES = 10

N_PAD = 10240
TRASH = N_PAD - 1
CHUNK = 128
NW = 32
CPT = 79
E_PAD = NW * CPT * CHUNK
NCHUNKS = E_PAD // CHUNK
ZROWS = 64
RPS = N_PAD // 16
DRNG = N_PAD // NW
LCAP = E_PAD + 14592
CAPB = 96
SLAB = 16
NSLAB = NCHUNKS // SLAB
FLUSH = 2048

_SC_PARAMS = pltpu.CompilerParams(use_tc_tiling_on_sc=False)
_SC_PARAMS_NL = pltpu.CompilerParams(
    use_tc_tiling_on_sc=False, needs_layout_passes=False)


def _mesh():
    return plsc.VectorSubcoreMesh(
        core_axis_name="c", subcore_axis_name="s", num_cores=2, num_subcores=16)


def _zero_fill(zbuf, d):
    zv = jnp.zeros((16,), jnp.float32)

    @pl.loop(0, ZROWS)
    def _(r):
        for c in range(d // 16):
            zbuf[r, pl.ds(c * 16, 16)] = zv


def _redirect(sidx, didx):
    trash_v = jnp.full((16,), TRASH, jnp.int32)
    for j in range(CHUNK // 16):
        s16 = sidx[pl.ds(j * 16, 16)]
        d16 = didx[pl.ds(j * 16, 16)]
        didx[pl.ds(j * 16, 16)] = jnp.where(s16 == d16, trash_v, d16)


@functools.cache
def _get_deg_kernel():
    return functools.partial(
        pl.kernel,
        mesh=_mesh(),
        compiler_params=_SC_PARAMS,
        out_type=jax.ShapeDtypeStruct((2, N_PAD, 16), jnp.float32),
        scratch_types=[
            pltpu.VMEM((2, CHUNK), jnp.int32),
            pltpu.VMEM((2, CHUNK), jnp.int32),
            pltpu.VMEM((CHUNK, 16), jnp.float32),
            pltpu.VMEM((ZROWS, 16), jnp.float32),
            pltpu.VMEM_SHARED((N_PAD, 16), jnp.float32),
        ],
    )(_deg_body)


def _deg_body(src_hbm, dst_hbm, out_hbm, sidx, didx, crows, zbuf, aggsh):
    core = lax.axis_index("c")
    sub = lax.axis_index("s")
    wid = core * 16 + sub

    _zero_fill(zbuf, 16)

    @pl.loop(0, RPS // ZROWS)
    def _(i):
        pltpu.sync_copy(zbuf, aggsh.at[pl.ds(sub * RPS + i * ZROWS, ZROWS)])

    e0 = jnp.where(
        lax.iota(jnp.int32, 16) == jnp.zeros((16,), jnp.int32),
        jnp.ones((16,), jnp.float32),
        jnp.zeros((16,), jnp.float32),
    )

    @pl.loop(0, CHUNK)
    def _(r):
        crows[r, pl.ds(0, 16)] = e0

    plsc.subcore_barrier()

    @pl.loop(0, CPT)
    def _(i):
        ci = wid * CPT + i
        pltpu.sync_copy(src_hbm.at[ci], sidx.at[0])
        pltpu.sync_copy(dst_hbm.at[ci], didx.at[0])
        _redirect(sidx.at[0], didx.at[0])
        pltpu.sync_copy(crows, aggsh.at[didx.at[0]], add=True)

    plsc.subcore_barrier()
    pltpu.sync_copy(
        aggsh.at[pl.ds(sub * RPS, RPS)],
        out_hbm.at[core, pl.ds(sub * RPS, RPS)],
    )


@functools.cache
def _get_scan_kernel():
    return functools.partial(
        pl.kernel,
        mesh=_mesh(),
        compiler_params=_SC_PARAMS_NL,
        out_type=[
            jax.ShapeDtypeStruct((NW, LCAP), jnp.int32),
            jax.ShapeDtypeStruct((NW, LCAP), jnp.int32),
            jax.ShapeDtypeStruct((NW, LCAP), jnp.float32),
            jax.ShapeDtypeStruct((NW, 16), jnp.int32),
        ],
        scratch_types=[
            pltpu.VMEM((SLAB, CHUNK), jnp.int32),
            pltpu.VMEM((SLAB, CHUNK), jnp.int32),
            pltpu.VMEM((FLUSH + CHUNK + 16,), jnp.int32),
            pltpu.VMEM((FLUSH + CHUNK + 16,), jnp.int32),
            pltpu.VMEM((N_PAD,), jnp.float32),
            pltpu.VMEM((CHUNK,), jnp.int32),
            pltpu.VMEM((CHUNK,), jnp.int32),
            pltpu.VMEM((CHUNK,), jnp.float32),
            pltpu.VMEM((16,), jnp.int32),
        ],
    )(_scan_body)


def _scan_body(src_hbm, dst_hbm, dinv_hbm,
               esrc_hbm, eloc_hbm, enrm_hbm, ecnt_hbm,
               sslab, dslab, vsrc, vdl, dinvb, nsrc, ndl, nnrm, ncnt):
    core = lax.axis_index("c")
    sub = lax.axis_index("s")
    wid = core * 16 + sub
    lo = wid * DRNG

    pltpu.sync_copy(dinv_hbm, dinvb)

    lo_v = jnp.full((16,), lo, jnp.int32)
    hi_v = jnp.full((16,), lo + DRNG, jnp.int32)

    def compact_step(g, cnt, wpos):
        s16 = sslab[g // 8, pl.ds((g % 8) * 16, 16)]
        d16 = dslab[g // 8, pl.ds((g % 8) * 16, 16)]
        m = (d16 >= lo_v) & (d16 < hi_v)
        npc = plsc.all_reduce_population_count(m)
        nmatch = jnp.max(npc)
        plsc.store_compressed(vsrc.at[pl.ds(cnt, 16)], s16, mask=m)
        plsc.store_compressed(vdl.at[pl.ds(cnt, 16)], d16 - lo_v, mask=m)
        return cnt + nmatch, wpos

    @pl.loop(0, NSLAB, init_carry=(jnp.int32(0), jnp.int32(0)))
    def outer(k, carry):
        cnt0, wpos0 = carry
        pltpu.sync_copy(src_hbm.at[pl.ds(pl.multiple_of(k * SLAB, SLAB), SLAB)], sslab)
        pltpu.sync_copy(dst_hbm.at[pl.ds(pl.multiple_of(k * SLAB, SLAB), SLAB)], dslab)

        @pl.loop(0, SLAB * 8, init_carry=(cnt0, wpos0))
        def inner(g, c):
            cnt, wpos = c
            cnt, wpos = compact_step(g, cnt, wpos)
            flush = cnt >= FLUSH

            @pl.when(flush)
            def _():
                wp = pl.multiple_of(wpos, FLUSH)
                pltpu.sync_copy(vsrc.at[pl.ds(0, FLUSH)],
                                esrc_hbm.at[wid, pl.ds(wp, FLUSH)])
                pltpu.sync_copy(vdl.at[pl.ds(0, FLUSH)],
                                eloc_hbm.at[wid, pl.ds(wp, FLUSH)])
                for j in range(2):
                    vsrc[pl.ds(j * 16, 16)] = vsrc[pl.ds(FLUSH + j * 16, 16)]
                    vdl[pl.ds(j * 16, 16)] = vdl[pl.ds(FLUSH + j * 16, 16)]

            cnt = jnp.where(flush, cnt - FLUSH, cnt)
            wpos = jnp.where(flush, wpos + FLUSH, wpos)
            return cnt, wpos

        return inner

    cnt, wpos = outer

    pad_src = jnp.full((16,), TRASH, jnp.int32)
    pad_dl = jnp.full((16,), DRNG, jnp.int32)
    for j in range(CHUNK // 16):
        vsrc[pl.ds(cnt + j * 16, 16)] = pad_src
        vdl[pl.ds(cnt + j * 16, 16)] = pad_dl
    tail = ((cnt + CHUNK - 1) // CHUNK) * CHUNK

    @pl.when(tail > 0)
    def _():
        wp = pl.multiple_of(wpos, FLUSH)
        pltpu.sync_copy(vsrc.at[pl.ds(0, FLUSH)],
                        esrc_hbm.at[wid, pl.ds(wp, FLUSH)])
        pltpu.sync_copy(vdl.at[pl.ds(0, FLUSH)],
                        eloc_hbm.at[wid, pl.ds(wp, FLUSH)])

    nb = (wpos + tail) // CHUNK
    ncnt[pl.ds(0, 16)] = jnp.full((16,), nb, jnp.int32)
    pltpu.sync_copy(ncnt, ecnt_hbm.at[wid])

    zero16 = jnp.zeros((16,), jnp.float32)
    maxn = jnp.full((16,), N_PAD - 1, jnp.int32)

    @pl.loop(0, nb)
    def _(b):
        bo = pl.multiple_of(b * CHUNK, CHUNK)
        pltpu.sync_copy(esrc_hbm.at[wid, pl.ds(bo, CHUNK)], nsrc)
        pltpu.sync_copy(eloc_hbm.at[wid, pl.ds(bo, CHUNK)], ndl)
        for j in range(CHUNK // 16):
            s16 = nsrc[pl.ds(j * 16, 16)]
            d16 = ndl[pl.ds(j * 16, 16)] + lo_v
            ds_ = plsc.load_gather(dinvb, [s16])
            dd_ = plsc.load_gather(dinvb, [jnp.minimum(d16, maxn)])
            nnrm[pl.ds(j * 16, 16)] = jnp.where(s16 == d16, zero16, ds_ * dd_)
        pltpu.sync_copy(nnrm, enrm_hbm.at[wid, pl.ds(bo, CHUNK)])


@functools.cache
def _make_acc_kernel(d):
    def _acc_body(esrc_hbm, eloc_hbm, enrm_hbm, ecnt_hbm, h_hbm, out_hbm,
                  lsrc, lloc, lnrm, rows0, rows1, acc, cbuf, gs0, gs1):
        core = lax.axis_index("c")
        sub = lax.axis_index("s")
        wid = core * 16 + sub
        lo = wid * DRNG

        @pl.loop(0, DRNG + 16)
        def _(r):
            for c in range(d // 16):
                acc[r, pl.ds(c * 16, 16)] = jnp.zeros((16,), jnp.float32)

        pltpu.sync_copy(ecnt_hbm.at[wid], cbuf)
        nb = jnp.max(cbuf[pl.ds(0, 16)])
        nsb = (nb + CAPB - 1) // CAPB

        def start_gather(b, rows, sem):
            io = pl.multiple_of(b * CHUNK, CHUNK)
            return pltpu.async_copy(
                h_hbm.at[lsrc.at[pl.ds(io, CHUNK)]], rows, sem)

        def wait_gather(b, rows, sem):
            io = pl.multiple_of(b * CHUNK, CHUNK)
            pltpu.make_async_copy(
                h_hbm.at[lsrc.at[pl.ds(io, CHUNK)]], rows, sem).wait()

        def process(b, rows):
            @pl.loop(0, CHUNK, unroll=4)
            def _(e):
                off = b * CHUNK + e
                nv = jnp.full((16,), lnrm[pl.ds(off, 16)][0], jnp.float32)
                dl = lloc[pl.ds(off, 16)][0]
                for c in range(d // 16):
                    upd = rows[e, pl.ds(c * 16, 16)] * nv
                    plsc.addupdate(acc.at[dl, pl.ds(c * 16, 16)], upd)

        @pl.loop(0, nsb)
        def _(sb):
            base = sb * CAPB
            rem = jnp.minimum(nb - base, CAPB)
            bo = pl.multiple_of(base * CHUNK, CHUNK)
            pltpu.sync_copy(esrc_hbm.at[wid, pl.ds(bo, CAPB * CHUNK)],
                            lsrc.at[pl.ds(0, CAPB * CHUNK)])
            pltpu.sync_copy(eloc_hbm.at[wid, pl.ds(bo, CAPB * CHUNK)],
                            lloc.at[pl.ds(0, CAPB * CHUNK)])
            pltpu.sync_copy(enrm_hbm.at[wid, pl.ds(bo, CAPB * CHUNK)],
                            lnrm.at[pl.ds(0, CAPB * CHUNK)])

            @pl.when(rem > 0)
            def _():
                start_gather(0, rows0, gs0)

            @pl.loop(0, rem, step=2)
            def _(b0):
                @pl.when(b0 + 1 < rem)
                def _():
                    start_gather(b0 + 1, rows1, gs1)

                wait_gather(b0, rows0, gs0)
                process(b0, rows0)

                @pl.when(b0 + 2 < rem)
                def _():
                    start_gather(b0 + 2, rows0, gs0)

                @pl.when(b0 + 1 < rem)
                def _():
                    wait_gather(b0 + 1, rows1, gs1)
                    process(b0 + 1, rows1)

        pltpu.sync_copy(acc.at[pl.ds(0, DRNG)], out_hbm.at[pl.ds(pl.multiple_of(lo, DRNG), DRNG)])

    return functools.partial(
        pl.kernel,
        mesh=_mesh(),
        compiler_params=_SC_PARAMS_NL,
        out_type=jax.ShapeDtypeStruct((N_PAD, d), jnp.float32),
        scratch_types=[
            pltpu.VMEM((CAPB * CHUNK + 16,), jnp.int32),
            pltpu.VMEM((CAPB * CHUNK + 16,), jnp.int32),
            pltpu.VMEM((CAPB * CHUNK + 16,), jnp.float32),
            pltpu.VMEM((CHUNK, d), jnp.float32),
            pltpu.VMEM((CHUNK, d), jnp.float32),
            pltpu.VMEM((DRNG + 16, d), jnp.float32),
            pltpu.VMEM((16,), jnp.int32),
            pltpu.SemaphoreType.DMA,
            pltpu.SemaphoreType.DMA,
        ],
    )(_acc_body)


@functools.cache
def _get_pool_kernel():
    return functools.partial(
        pl.kernel,
        mesh=_mesh(),
        compiler_params=_SC_PARAMS_NL,
        out_type=jax.ShapeDtypeStruct((G * K, 128), jnp.float32),
        scratch_types=[
            pltpu.VMEM((N_PAD,), jnp.int32),
            pltpu.VMEM((N_PAD,), jnp.float32),
            pltpu.VMEM((64,), jnp.int32),
            pltpu.VMEM((64, 128), jnp.float32),
            pltpu.SemaphoreType.DMA,
        ],
    )(_pool_body)


def _pool_body(batch_hbm, lastv_hbm, xc_hbm, out_hbm, bbuf, vbuf, slotbuf, rows, sem):
    core = lax.axis_index("c")
    sub = lax.axis_index("s")
    wid = core * 16 + sub
    g0 = wid * 2

    pltpu.sync_copy(batch_hbm, bbuf)
    pltpu.sync_copy(lastv_hbm, vbuf)

    trash_v = jnp.full((16,), TRASH, jnp.int32)
    for j in range(4):
        slotbuf[pl.ds(16 * j, 16)] = trash_v

    iota16 = lax.iota(jnp.int32, 16)
    zero16 = jnp.zeros((16,), jnp.int32)
    one16 = jnp.ones((16,), jnp.int32)
    g0v = jnp.full((16,), g0, jnp.int32)
    g1v = g0v + one16
    k16 = jnp.full((16,), K, jnp.int32)

    @pl.loop(0, N_PAD // 16, init_carry=(zero16, zero16, zero16))
    def counts(j, c):
        s0v, n0v, n1v = c
        b16 = bbuf[pl.ds(j * 16, 16)]
        lt = plsc.all_reduce_population_count(b16 < g0v)
        e0 = plsc.all_reduce_population_count(b16 == g0v)
        e1 = plsc.all_reduce_population_count(b16 == g1v)
        return (s0v + lt, n0v + e0, n1v + e1)

    s0v, n0v, n1v = counts
    s0 = jnp.max(s0v)
    n0 = jnp.max(n0v)
    n1 = jnp.max(n1v)

    for gl in range(2):
        start = s0 if gl == 0 else s0 + n0
        n = n0 if gl == 0 else n1
        nv = jnp.full((16,), n, jnp.int32)
        base_slot = jnp.full((16,), gl * K, jnp.int32)

        @pl.loop(0, (n + 15) // 16)
        def _(ic):
            bi = ic * 16
            ivec = iota16 + jnp.full((16,), bi, jnp.int32)
            vi = vbuf[pl.ds(start + bi, 16)]

            @pl.loop(0, n, init_carry=zero16)
            def rank_loop(j, rank):
                vj = vbuf[pl.ds(start + j, 16)][0]
                vjv = jnp.full((16,), vj, jnp.float32)
                jv = jnp.full((16,), j, jnp.int32)
                beats = (vjv > vi) | ((vjv == vi) & (jv < ivec))
                return rank + jnp.where(beats, one16, zero16)

            rank = rank_loop
            mask = (rank < k16) & (ivec < nv)
            node = jnp.full((16,), start + bi, jnp.int32) + iota16
            plsc.store_scatter(slotbuf, [base_slot + rank], node, mask=mask)

    pltpu.async_copy(xc_hbm.at[slotbuf], rows, sem).wait()
    pltpu.sync_copy(rows.at[pl.ds(0, 2 * K)], out_hbm.at[pl.ds(wid * 2 * K, 2 * K)])


def _prep_body(degP_ref, x_ref, w1_ref, dinv_ref, invdeg_ref, h1_ref):
    deg = degP_ref[0, :, 0:1] + degP_ref[1, :, 0:1] + 1.0
    dinv_ref[...] = lax.rsqrt(deg)
    invdeg_ref[...] = 1.0 / deg
    h1_ref[...] = jnp.dot(x_ref[...], w1_ref[...], preferred_element_type=jnp.float32)


def _tc_prep(degP, xpad, W1):
    return pl.pallas_call(
        _prep_body,
        out_shape=[
            jax.ShapeDtypeStruct((N_PAD, 1), jnp.float32),
            jax.ShapeDtypeStruct((N_PAD, 1), jnp.float32),
            jax.ShapeDtypeStruct((N_PAD, 32), jnp.float32),
        ],
    )(degP, xpad, W1)


def _layer_body(w, wn):
    def body(acc_ref, h_ref, invdeg_ref, b_ref, *rest):
        if wn:
            wn_ref, xl_ref, hn_ref = rest
        else:
            (xl_ref,) = rest
        rows = lax.broadcasted_iota(jnp.int32, (N_PAD, 1), 0)
        valid = jnp.where(rows < N, 1.0, 0.0)
        t = acc_ref[:, :w] + invdeg_ref[...] * h_ref[...]
        xl = jnp.tanh(t + b_ref[...]) * valid
        xl_ref[...] = xl
        if wn:
            hn_ref[...] = jnp.dot(xl, wn_ref[...], preferred_element_type=jnp.float32)
    return body


def _tc_layer(acc, h, invdeg, b, Wn):
    w = h.shape[1]
    return pl.pallas_call(
        _layer_body(w, True),
        out_shape=[
            jax.ShapeDtypeStruct((N_PAD, w), jnp.float32),
            jax.ShapeDtypeStruct((N_PAD, Wn.shape[1]), jnp.float32),
        ],
    )(acc, h, invdeg, b.reshape(1, w), Wn)


_PACK_ROWS = 1280


def _layer4_pack_body(acc_ref, h_ref, invdeg_ref, b_ref,
                      x1_ref, x2_ref, x3_ref, xc_ref, lastv_ref):
    rows = (lax.broadcasted_iota(jnp.int32, (_PACK_ROWS, 1), 0)
            + pl.program_id(0) * _PACK_ROWS)
    valid = jnp.where(rows < N, 1.0, 0.0)
    t = acc_ref[:, 0:1] + invdeg_ref[...] * h_ref[...]
    x4 = jnp.tanh(t + b_ref[...]) * valid
    z = jnp.zeros((_PACK_ROWS, 31), jnp.float32)
    xc_ref[...] = jnp.concatenate(
        [x1_ref[...], x2_ref[...], x3_ref[...], x4, z], axis=1)
    lastv_ref[...] = x4


def _tc_layer4_pack(acc, h, invdeg, b, x1, x2, x3):
    nb = N_PAD // _PACK_ROWS
    row_spec = lambda w: pl.BlockSpec((_PACK_ROWS, w), lambda i: (i, 0))
    return pl.pallas_call(
        _layer4_pack_body,
        grid=(nb,),
        in_specs=[
            row_spec(16),
            row_spec(1),
            row_spec(1),
            pl.BlockSpec((1, 1), lambda i: (0, 0)),
            row_spec(32),
            row_spec(32),
            row_spec(32),
        ],
        out_specs=[row_spec(128), row_spec(1)],
        out_shape=[
            jax.ShapeDtypeStruct((N_PAD, 128), jnp.float32),
            jax.ShapeDtypeStruct((N_PAD, 1), jnp.float32),
        ],
    )(acc, h, invdeg, b.reshape(1, 1), x1, x2, x3)


def _tail_body(dense_ref, w5_ref, b5_ref, w6_ref, b6_ref, wc1_ref, bc1_ref,
               wc2_ref, bc2_ref, out_ref):
    t1 = jnp.dot(dense_ref[...], w5_ref[...], preferred_element_type=jnp.float32)
    t1 = jax.nn.relu(t1 + b5_ref[...])
    t3 = t1.reshape(G * K // 2, 2, 16)
    p = jnp.maximum(t3[:, 0, :], t3[:, 1, :])
    p3 = p.reshape(G, 15, 16)
    blocks = []
    for pp in range(11):
        accp = jnp.zeros((G, 32), jnp.float32)
        for k in range(5):
            accp = accp + jnp.dot(p3[:, pp + k, :], w6_ref[k],
                                  preferred_element_type=jnp.float32)
        blocks.append(accp)
    acc = jnp.concatenate(blocks, axis=1)
    acc = jax.nn.relu(acc + jnp.tile(b6_ref[...], (1, 11)))
    h = jax.nn.relu(jnp.dot(acc, wc1_ref[...], preferred_element_type=jnp.float32) + bc1_ref[...])
    out_ref[...] = jnp.dot(h, wc2_ref[...], preferred_element_type=jnp.float32) + bc2_ref[...]


def _tc_tail(dense, W5t, b5, W6k, b6, Wc1r, bc1, Wc2, bc2):
    return pl.pallas_call(
        _tail_body,
        out_shape=jax.ShapeDtypeStruct((G, NUM_CLASSES), jnp.float32),
    )(dense, W5t, b5.reshape(1, 16), W6k, b6.reshape(1, 32), Wc1r,
      bc1.reshape(1, 256), Wc2, bc2.reshape(1, NUM_CLASSES))


def kernel(x, edge_index, batch, W1, b1, W2, b2, W3, b3, W4, b4, W5, b5, W6, b6, Wc1, bc1, Wc2, bc2):
    src = edge_index[0]
    dst = edge_index[1]
    epad = jnp.full((E_PAD - src.shape[0],), TRASH, jnp.int32)
    src_p = jnp.concatenate([src, epad]).reshape(NCHUNKS, CHUNK)
    dst_p = jnp.concatenate([dst, epad]).reshape(NCHUNKS, CHUNK)
    xpad = jnp.zeros((N_PAD, 128), x.dtype).at[:N].set(x)
    batch_pad = jnp.concatenate([batch, jnp.full((N_PAD - N,), 127, jnp.int32)])
    W5t = jnp.zeros((128, 16), W5.dtype).at[:97, :].set(W5[:, 0, :].T)
    W6k = W6.transpose(2, 1, 0)
    Wc1r = Wc1.reshape(32, 11, 256).transpose(1, 0, 2).reshape(352, 256)

    degP = _get_deg_kernel()(src_p, dst_p)
    dinv, invdeg, h1 = _tc_prep(degP, xpad, W1)

    esrc, eloc, enrm, ecnt = _get_scan_kernel()(src_p, dst_p, dinv.reshape(N_PAD))

    acck32 = _make_acc_kernel(32)
    A1 = acck32(esrc, eloc, enrm, ecnt, h1)
    x1, h2 = _tc_layer(A1, h1, invdeg, b1, W2)
    A2 = acck32(esrc, eloc, enrm, ecnt, h2)
    x2, h3 = _tc_layer(A2, h2, invdeg, b2, W3)
    A3 = acck32(esrc, eloc, enrm, ecnt, h3)
    x3, h4 = _tc_layer(A3, h3, invdeg, b3, W4)
    h4p = jnp.zeros((N_PAD, 16), jnp.float32).at[:, :1].set(h4)
    A4 = _make_acc_kernel(16)(esrc, eloc, enrm, ecnt, h4p)
    xc, lastv = _tc_layer4_pack(A4, h4, invdeg, b4, x1, x2, x3)

    dense = _get_pool_kernel()(batch_pad, lastv.reshape(N_PAD), xc)
    return _tc_tail(dense, W5t, b5, W6k, b6, Wc1r, bc1, Wc2, bc2)
